# baseline (speedup 1.0000x reference)
import functools

import jax
import jax.numpy as jnp
from jax import lax
from jax.experimental import pallas as pl
from jax.experimental.pallas import tpu as pltpu
from jax.experimental.pallas import tpu_sc as plsc

BINS = 16
LO = 1.0
HI = 80.0
WIDTH = (HI - LO) / BINS
B, N, C = 4, 2048, 3
FEAT = 128
HID = FEAT // 2

NW = 32
RW = B * N // NW
WPB = N // RW
GROUPS = RW // 16


def _sc_hist_body(xt_hbm, counts_hbm, x0_v, x1_v, x2_v, cnt_v):
    wid = lax.axis_index("s") * 2 + lax.axis_index("c")
    b = wid // WPB
    i_off = (wid % WPB) * RW

    pltpu.sync_copy(xt_hbm.at[pl.ds((b * C + 0) * N, N)], x0_v)
    pltpu.sync_copy(xt_hbm.at[pl.ds((b * C + 1) * N, N)], x1_v)
    pltpu.sync_copy(xt_hbm.at[pl.ds((b * C + 2) * N, N)], x2_v)

    zeros16 = jnp.zeros((16,), jnp.float32)

    def zrow(r, carry):
        cnt_v[pl.ds(r * 16, 16)] = zeros16
        return carry

    lax.fori_loop(0, RW * BINS // 16, zrow, 0)

    lane = lax.iota(jnp.int32, 16)
    ones = jnp.ones((16,), jnp.float32)

    def group_body(g, carry):
        base = i_off + g * 16
        xi0 = x0_v[pl.ds(base, 16)]
        xi1 = x1_v[pl.ds(base, 16)]
        xi2 = x2_v[pl.ds(base, 16)]
        rowbase = (g * 16 + lane) * BINS

        def jv_body(jv, c):
            xj0 = x0_v[pl.ds(jv * 16, 16)]
            xj1 = x1_v[pl.ds(jv * 16, 16)]
            xj2 = x2_v[pl.ds(jv * 16, 16)]
            for l in range(16):
                d0 = xi0 - xj0[l]
                d1 = xi1 - xj1[l]
                d2 = xi2 - xj2[l]
                sq = jnp.maximum(d0 * d0 + d1 * d1 + d2 * d2, 1e-24)
                u = lax.bitcast_convert_type(sq, jnp.int32)
                y = lax.bitcast_convert_type((u >> 1) + 0x1FBD1DF5, jnp.float32)
                y = 0.5 * (y + sq / y)
                y = 0.5 * (y + sq / y)
                y = 0.5 * (y + sq / y)
                idx = ((y - LO) / WIDTH).astype(jnp.int32)
                idx = jnp.minimum(jnp.maximum(idx, 0), BINS - 1)
                valid = (y >= LO) & (y <= HI)
                plsc.addupdate_scatter(cnt_v, [rowbase + idx], ones, mask=valid)
            return c

        lax.fori_loop(0, N // 16, jv_body, 0)
        return carry

    lax.fori_loop(0, GROUPS, group_body, 0)

    pltpu.sync_copy(cnt_v, counts_hbm.at[pl.ds(wid * RW * BINS, RW * BINS)])


def _histograms(x):
    xt = jnp.transpose(x, (0, 2, 1)).reshape(-1)
    f = functools.partial(
        pl.kernel,
        out_type=jax.ShapeDtypeStruct((B * N * BINS,), jnp.float32),
        mesh=plsc.VectorSubcoreMesh(core_axis_name="c", subcore_axis_name="s"),
        compiler_params=pltpu.CompilerParams(needs_layout_passes=False),
        scratch_types=[
            pltpu.VMEM((N,), jnp.float32),
            pltpu.VMEM((N,), jnp.float32),
            pltpu.VMEM((N,), jnp.float32),
            pltpu.VMEM((RW * BINS,), jnp.float32),
        ],
    )(_sc_hist_body)
    return f(xt).reshape(B * N, BINS)


def _mlp_body(counts_ref, fea_ref, W1_ref, b1_ref, g1_ref, be1_ref,
              W2_ref, b2_ref, g2_ref, be2_ref, W3_ref, b3_ref, out_ref):
    counts = counts_ref[...]
    s = jnp.sum(counts, axis=1, keepdims=True)
    hist = counts / s

    def bn(z, g, be):
        m = jnp.mean(z, axis=1, keepdims=True)
        v = jnp.mean((z - m) * (z - m), axis=1, keepdims=True)
        return (z - m) / jnp.sqrt(v + 1e-5) * g + be

    z1 = jax.lax.dot_general(W1_ref[...], hist, (((1,), (1,)), ((), ())),
                             preferred_element_type=jnp.float32) + b1_ref[...]
    h1 = jax.nn.relu(bn(z1, g1_ref[...], be1_ref[...]))
    z2 = jax.lax.dot_general(W2_ref[...], h1, (((1,), (0,)), ((), ())),
                             preferred_element_type=jnp.float32) + b2_ref[...]
    h2 = jax.nn.relu(bn(z2, g2_ref[...], be2_ref[...]))
    z3 = jax.lax.dot_general(W3_ref[...], h2, (((1,), (0,)), ((), ())),
                             preferred_element_type=jnp.float32) + b3_ref[...]
    for b in range(B):
        out_ref[b] = fea_ref[b] + z3[:, b * N:(b + 1) * N]


def kernel(x, fea, W1, b1, g1, be1, W2, b2, g2, be2, W3, b3):
    counts = _histograms(x)
    out = pl.pallas_call(
        _mlp_body,
        out_shape=jax.ShapeDtypeStruct((B, FEAT, N), jnp.float32),
    )(counts, fea, W1, b1.reshape(HID, 1), g1.reshape(HID, 1),
      be1.reshape(HID, 1), W2, b2.reshape(HID, 1), g2.reshape(HID, 1),
      be2.reshape(HID, 1), W3, b3.reshape(FEAT, 1))
    return out


# speedup vs baseline: 1.9724x; 1.9724x over previous
import functools

import jax
import jax.numpy as jnp
from jax import lax
from jax.experimental import pallas as pl
from jax.experimental.pallas import tpu as pltpu
from jax.experimental.pallas import tpu_sc as plsc

BINS = 16
LO = 1.0
HI = 80.0
WIDTH = (HI - LO) / BINS
INVW = BINS / (HI - LO)
B, N, C = 4, 2048, 3
FEAT = 128
HID = FEAT // 2

NW = 32
RW = B * N // NW
WPB = N // RW
GROUPS = RW // 16


def _sc_hist_body(xt_hbm, counts_hbm, x0_v, x1_v, x2_v, cnt_v):
    wid = lax.axis_index("s") * 2 + lax.axis_index("c")
    b = wid // WPB
    i_off = (wid % WPB) * RW

    pltpu.sync_copy(xt_hbm.at[pl.ds((b * C + 0) * N, N)], x0_v)
    pltpu.sync_copy(xt_hbm.at[pl.ds((b * C + 1) * N, N)], x1_v)
    pltpu.sync_copy(xt_hbm.at[pl.ds((b * C + 2) * N, N)], x2_v)

    zeros16 = jnp.zeros((16,), jnp.float32)

    def zrow(r, carry):
        cnt_v[pl.ds(r * 16, 16)] = zeros16
        return carry

    lax.fori_loop(0, RW * BINS // 16, zrow, 0)

    lane = lax.iota(jnp.int32, 16)
    ones = jnp.ones((16,), jnp.float32)

    def group_body(g, carry):
        base = i_off + g * 16
        xi0 = x0_v[pl.ds(base, 16)]
        xi1 = x1_v[pl.ds(base, 16)]
        xi2 = x2_v[pl.ds(base, 16)]
        rowbase = (g * 16 + lane) * BINS

        def jv_body(jv, c):
            xj0 = x0_v[pl.ds(jv * 16, 16)]
            xj1 = x1_v[pl.ds(jv * 16, 16)]
            xj2 = x2_v[pl.ds(jv * 16, 16)]
            for l in range(16):
                d0 = xi0 - xj0[l]
                d1 = xi1 - xj1[l]
                d2 = xi2 - xj2[l]
                sq = jnp.maximum(d0 * d0 + d1 * d1 + d2 * d2, 1e-24)
                u = lax.bitcast_convert_type(sq, jnp.int32)
                r = lax.bitcast_convert_type(0x5F3759DF - (u >> 1), jnp.float32)
                r = r * (1.5 - 0.5 * sq * r * r)
                r = r * (1.5 - 0.5 * sq * r * r)
                r = r * (1.5 - 0.5 * sq * r * r)
                y = sq * r
                idx = ((y - LO) * INVW).astype(jnp.int32)
                idx = jnp.minimum(jnp.maximum(idx, 0), BINS - 1)
                valid = (y >= LO) & (y <= HI)
                plsc.addupdate_scatter(cnt_v, [rowbase + idx], ones, mask=valid)
            return c

        lax.fori_loop(0, N // 16, jv_body, 0)
        return carry

    lax.fori_loop(0, GROUPS, group_body, 0)

    pltpu.sync_copy(cnt_v, counts_hbm.at[pl.ds(wid * RW * BINS, RW * BINS)])


def _histograms(x):
    xt = jnp.transpose(x, (0, 2, 1)).reshape(-1)
    f = functools.partial(
        pl.kernel,
        out_type=jax.ShapeDtypeStruct((B * N * BINS,), jnp.float32),
        mesh=plsc.VectorSubcoreMesh(core_axis_name="c", subcore_axis_name="s"),
        compiler_params=pltpu.CompilerParams(needs_layout_passes=False),
        scratch_types=[
            pltpu.VMEM((N,), jnp.float32),
            pltpu.VMEM((N,), jnp.float32),
            pltpu.VMEM((N,), jnp.float32),
            pltpu.VMEM((RW * BINS,), jnp.float32),
        ],
    )(_sc_hist_body)
    return f(xt).reshape(B * N, BINS)


def _mlp_body(counts_ref, fea_ref, W1_ref, b1_ref, g1_ref, be1_ref,
              W2_ref, b2_ref, g2_ref, be2_ref, W3_ref, b3_ref, out_ref):
    counts = counts_ref[...]
    s = jnp.sum(counts, axis=1, keepdims=True)
    hist = counts / s

    def bn(z, g, be):
        m = jnp.mean(z, axis=1, keepdims=True)
        v = jnp.mean((z - m) * (z - m), axis=1, keepdims=True)
        return (z - m) / jnp.sqrt(v + 1e-5) * g + be

    z1 = jax.lax.dot_general(W1_ref[...], hist, (((1,), (1,)), ((), ())),
                             preferred_element_type=jnp.float32) + b1_ref[...]
    h1 = jax.nn.relu(bn(z1, g1_ref[...], be1_ref[...]))
    z2 = jax.lax.dot_general(W2_ref[...], h1, (((1,), (0,)), ((), ())),
                             preferred_element_type=jnp.float32) + b2_ref[...]
    h2 = jax.nn.relu(bn(z2, g2_ref[...], be2_ref[...]))
    z3 = jax.lax.dot_general(W3_ref[...], h2, (((1,), (0,)), ((), ())),
                             preferred_element_type=jnp.float32) + b3_ref[...]
    for b in range(B):
        out_ref[b] = fea_ref[b] + z3[:, b * N:(b + 1) * N]


def kernel(x, fea, W1, b1, g1, be1, W2, b2, g2, be2, W3, b3):
    counts = _histograms(x)
    out = pl.pallas_call(
        _mlp_body,
        out_shape=jax.ShapeDtypeStruct((B, FEAT, N), jnp.float32),
    )(counts, fea, W1, b1.reshape(HID, 1), g1.reshape(HID, 1),
      be1.reshape(HID, 1), W2, b2.reshape(HID, 1), g2.reshape(HID, 1),
      be2.reshape(HID, 1), W3, b3.reshape(FEAT, 1))
    return out
